# Initial kernel scaffold; baseline (speedup 1.0000x reference)
import jax
import jax.numpy as jnp
from jax.experimental import pallas as pl


def kernel(nodes, neigh_index, features, W1, b1):
    raise NotImplementedError("write your pallas kernel here")


# speedup vs baseline: 6.1049x; 6.1049x over previous
import functools

import jax
import jax.numpy as jnp
from jax import lax
from jax.experimental import pallas as pl
from jax.experimental.pallas import tpu as pltpu
from jax.experimental.pallas import tpu_sc as plsc

B = 16384
DEG = 32
D = 128
LANES = 16
NW = 32
NB_W = B // NW
CHUNK = 128
ITEMS_PER_CHUNK = CHUNK // DEG
CHUNKS_W = NB_W * DEG // CHUNK
N_STAGE = NB_W // CHUNK
CHUNKS_PER_STAGE = CHUNKS_W // N_STAGE


def _sc_body(nodes_hbm, neigh_hbm, feat_hbm, self_out, neigh_out,
             nidx_v, sidx_v, rows_v, srows_v, stage_v, sem, sem2):
    wid = lax.axis_index("s") * 2 + lax.axis_index("c")
    pltpu.sync_copy(neigh_hbm.at[pl.ds(wid * CHUNKS_W, CHUNKS_W)], nidx_v)
    pltpu.sync_copy(nodes_hbm.at[pl.ds(wid * N_STAGE, N_STAGE)], sidx_v)
    out_base = wid * NB_W

    for q in range(N_STAGE):
        pltpu.async_copy(feat_hbm.at[sidx_v.at[q]], srows_v, sem2).wait()
        pltpu.sync_copy(srows_v, self_out.at[pl.ds(out_base + q * CHUNK, CHUNK)])

        def chunk_body(j2, _, q=q):
            c = q * CHUNKS_PER_STAGE + j2
            pltpu.async_copy(feat_hbm.at[nidx_v.at[c]], rows_v, sem).wait()
            for i in range(ITEMS_PER_CHUNK):
                base_r = i * DEG
                accs = tuple(rows_v[base_r, pl.ds(d * LANES, LANES)]
                             for d in range(D // LANES))

                def add_body(k, acc, base_r=base_r):
                    return tuple(acc[d] + rows_v[base_r + k, pl.ds(d * LANES, LANES)]
                                 for d in range(D // LANES))

                accs = lax.fori_loop(1, DEG, add_body, accs)
                r = j2 * ITEMS_PER_CHUNK + i
                for d in range(D // LANES):
                    stage_v[r, pl.ds(d * LANES, LANES)] = accs[d]
            return 0

        lax.fori_loop(0, CHUNKS_PER_STAGE, chunk_body, 0)
        pltpu.sync_copy(stage_v, neigh_out.at[pl.ds(out_base + q * CHUNK, CHUNK)])


_sc_gather_mean = functools.partial(
    pl.kernel,
    out_type=(jax.ShapeDtypeStruct((B, D), jnp.float32),
              jax.ShapeDtypeStruct((B, D), jnp.float32)),
    mesh=plsc.VectorSubcoreMesh(core_axis_name="c", subcore_axis_name="s"),
    scratch_types=[
        pltpu.VMEM((CHUNKS_W, CHUNK), jnp.int32),
        pltpu.VMEM((N_STAGE, CHUNK), jnp.int32),
        pltpu.VMEM((CHUNK, D), jnp.float32),
        pltpu.VMEM((CHUNK, D), jnp.float32),
        pltpu.VMEM((CHUNK, D), jnp.float32),
        pltpu.SemaphoreType.DMA,
        pltpu.SemaphoreType.DMA,
    ],
)(_sc_body)


def _tc_head(self_feats, neigh_sum, W1, b1):
    BB = 2048

    def mm(self_ref, neigh_ref, w_ref, b_ref, o_ref):
        s = self_ref[...]
        n = neigh_ref[...] * (1.0 / DEG)
        y = jnp.dot(s, w_ref[0:D, :], preferred_element_type=jnp.float32,
                    precision=lax.Precision.HIGHEST)
        y = y + jnp.dot(n, w_ref[D:2 * D, :], preferred_element_type=jnp.float32,
                        precision=lax.Precision.HIGHEST)
        y = y + b_ref[...]
        o_ref[...] = jnp.maximum(y, 0.0)

    return pl.pallas_call(
        mm,
        grid=(B // BB,),
        in_specs=[
            pl.BlockSpec((BB, D), lambda i: (i, 0)),
            pl.BlockSpec((BB, D), lambda i: (i, 0)),
            pl.BlockSpec((2 * D, D), lambda i: (0, 0)),
            pl.BlockSpec((1, D), lambda i: (0, 0)),
        ],
        out_specs=pl.BlockSpec((BB, D), lambda i: (i, 0)),
        out_shape=jax.ShapeDtypeStruct((B, D), jnp.float32),
    )(self_feats, neigh_sum, W1, b1.reshape(1, D))


def kernel(nodes, neigh_index, features, W1, b1):
    nodes2d = nodes.astype(jnp.int32).reshape(B // CHUNK, CHUNK)
    neigh2d = neigh_index.astype(jnp.int32).reshape(B * DEG // CHUNK, CHUNK)
    self_feats, neigh_sum = _sc_gather_mean(nodes2d, neigh2d, features)
    return _tc_head(self_feats, neigh_sum, W1, b1)


# speedup vs baseline: 9.8364x; 1.6112x over previous
import functools

import jax
import jax.numpy as jnp
from jax import lax
from jax.experimental import pallas as pl
from jax.experimental.pallas import tpu as pltpu
from jax.experimental.pallas import tpu_sc as plsc

B = 16384
DEG = 32
D = 128
LANES = 16
NW = 32
NB_W = B // NW
CHUNK = 128
ITEMS_PER_CHUNK = CHUNK // DEG
CHUNKS_W = NB_W * DEG // CHUNK
N_STAGE = NB_W // CHUNK
CHUNKS_PER_STAGE = CHUNKS_W // N_STAGE


def _sc_body(nodes_hbm, neigh_hbm, feat_hbm, self_out, neigh_out,
             nidx_v, sidx_v, rows0_v, rows1_v, srows_v, stage_v,
             sem0, sem1, sem2):
    wid = lax.axis_index("s") * 2 + lax.axis_index("c")
    pltpu.sync_copy(neigh_hbm.at[pl.ds(wid * CHUNKS_W, CHUNKS_W)], nidx_v)
    pltpu.sync_copy(nodes_hbm.at[pl.ds(wid * N_STAGE, N_STAGE)], sidx_v)
    out_base = wid * NB_W

    def start(c, rows, sem):
        pltpu.async_copy(feat_hbm.at[nidx_v.at[c]], rows, sem)

    def finish(c, rows, sem):
        pltpu.make_async_copy(feat_hbm.at[nidx_v.at[c]], rows, sem).wait()

    def reduce_chunk(rows, stage_base):
        for i in range(ITEMS_PER_CHUNK):
            base_r = i * DEG
            accs = tuple(rows[base_r, pl.ds(d * LANES, LANES)]
                         for d in range(D // LANES))

            def add_body(k, acc, base_r=base_r, rows=rows):
                return tuple(acc[d] + rows[base_r + k, pl.ds(d * LANES, LANES)]
                             for d in range(D // LANES))

            accs = lax.fori_loop(1, DEG, add_body, accs)
            for d in range(D // LANES):
                stage_v[stage_base + i, pl.ds(d * LANES, LANES)] = accs[d]

    start(0, rows0_v, sem0)
    start(1, rows1_v, sem1)

    for q in range(N_STAGE):
        pltpu.async_copy(feat_hbm.at[sidx_v.at[q]], srows_v, sem2).wait()
        pltpu.sync_copy(srows_v, self_out.at[pl.ds(out_base + q * CHUNK, CHUNK)])

        def pair_body(j, _, q=q):
            local = j - q * (CHUNKS_PER_STAGE // 2)
            c0 = 2 * j
            c1 = 2 * j + 1
            finish(c0, rows0_v, sem0)
            reduce_chunk(rows0_v, local * 2 * ITEMS_PER_CHUNK)

            @pl.when(c0 + 2 < CHUNKS_W)
            def _():
                start(c0 + 2, rows0_v, sem0)

            finish(c1, rows1_v, sem1)
            reduce_chunk(rows1_v, (local * 2 + 1) * ITEMS_PER_CHUNK)

            @pl.when(c1 + 2 < CHUNKS_W)
            def _():
                start(c1 + 2, rows1_v, sem1)

            return 0

        half = CHUNKS_PER_STAGE // 2
        lax.fori_loop(q * half, (q + 1) * half, pair_body, 0)
        pltpu.sync_copy(stage_v, neigh_out.at[pl.ds(out_base + q * CHUNK, CHUNK)])


_sc_gather_mean = functools.partial(
    pl.kernel,
    out_type=(jax.ShapeDtypeStruct((B, D), jnp.float32),
              jax.ShapeDtypeStruct((B, D), jnp.float32)),
    mesh=plsc.VectorSubcoreMesh(core_axis_name="c", subcore_axis_name="s"),
    scratch_types=[
        pltpu.VMEM((CHUNKS_W, CHUNK), jnp.int32),
        pltpu.VMEM((N_STAGE, CHUNK), jnp.int32),
        pltpu.VMEM((CHUNK, D), jnp.float32),
        pltpu.VMEM((CHUNK, D), jnp.float32),
        pltpu.VMEM((CHUNK, D), jnp.float32),
        pltpu.VMEM((CHUNK, D), jnp.float32),
        pltpu.SemaphoreType.DMA,
        pltpu.SemaphoreType.DMA,
        pltpu.SemaphoreType.DMA,
    ],
)(_sc_body)


def _tc_head(self_feats, neigh_sum, W1, b1):
    BB = 2048

    def mm(self_ref, neigh_ref, w_ref, b_ref, o_ref):
        s = self_ref[...]
        n = neigh_ref[...] * (1.0 / DEG)
        y = jnp.dot(s, w_ref[0:D, :], preferred_element_type=jnp.float32,
                    precision=lax.Precision.HIGHEST)
        y = y + jnp.dot(n, w_ref[D:2 * D, :], preferred_element_type=jnp.float32,
                        precision=lax.Precision.HIGHEST)
        y = y + b_ref[...]
        o_ref[...] = jnp.maximum(y, 0.0)

    return pl.pallas_call(
        mm,
        grid=(B // BB,),
        in_specs=[
            pl.BlockSpec((BB, D), lambda i: (i, 0)),
            pl.BlockSpec((BB, D), lambda i: (i, 0)),
            pl.BlockSpec((2 * D, D), lambda i: (0, 0)),
            pl.BlockSpec((1, D), lambda i: (0, 0)),
        ],
        out_specs=pl.BlockSpec((BB, D), lambda i: (i, 0)),
        out_shape=jax.ShapeDtypeStruct((B, D), jnp.float32),
    )(self_feats, neigh_sum, W1, b1.reshape(1, D))


def kernel(nodes, neigh_index, features, W1, b1):
    nodes2d = nodes.astype(jnp.int32).reshape(B // CHUNK, CHUNK)
    neigh2d = neigh_index.astype(jnp.int32).reshape(B * DEG // CHUNK, CHUNK)
    self_feats, neigh_sum = _sc_gather_mean(nodes2d, neigh2d, features)
    return _tc_head(self_feats, neigh_sum, W1, b1)


# speedup vs baseline: 9.9540x; 1.0119x over previous
import functools

import jax
import jax.numpy as jnp
from jax import lax
from jax.experimental import pallas as pl
from jax.experimental.pallas import tpu as pltpu
from jax.experimental.pallas import tpu_sc as plsc

B = 16384
DEG = 32
D = 128
LANES = 16
NW = 32
NB_W = B // NW
CHUNK = 128
ITEMS_PER_CHUNK = CHUNK // DEG
CHUNKS_W = NB_W * DEG // CHUNK
N_STAGE = NB_W // CHUNK
CHUNKS_PER_STAGE = CHUNKS_W // N_STAGE


def _sc_body(nodes_hbm, neigh_hbm, feat_hbm, self_out, neigh_out,
             nidx_v, sidx_v, rows0_v, rows1_v, srows_v, stage_v,
             sem0, sem1, sem2):
    wid = lax.axis_index("s") * 2 + lax.axis_index("c")
    pltpu.sync_copy(neigh_hbm.at[pl.ds(wid * CHUNKS_W, CHUNKS_W)], nidx_v)
    pltpu.sync_copy(nodes_hbm.at[pl.ds(wid * N_STAGE, N_STAGE)], sidx_v)
    out_base = wid * NB_W

    def start(c, rows, sem):
        pltpu.async_copy(feat_hbm.at[nidx_v.at[c]], rows, sem)

    def finish(c, rows, sem):
        pltpu.make_async_copy(feat_hbm.at[nidx_v.at[c]], rows, sem).wait()

    def row_vals(rows, r):
        return tuple(rows[r, pl.ds(d * LANES, LANES)]
                     for d in range(D // LANES))

    def reduce_chunk(rows, stage_base):
        for i in range(ITEMS_PER_CHUNK):
            base_r = i * DEG
            v0 = row_vals(rows, base_r)
            v1 = row_vals(rows, base_r + 1)
            accs = tuple(v0[d] + v1[d] for d in range(D // LANES))

            def add_body(k2, acc, base_r=base_r, rows=rows):
                r = base_r + 2 * k2
                va = row_vals(rows, r)
                vb = row_vals(rows, r + 1)
                return tuple(acc[d] + (va[d] + vb[d])
                             for d in range(D // LANES))

            accs = lax.fori_loop(1, DEG // 2, add_body, accs)
            for d in range(D // LANES):
                stage_v[stage_base + i, pl.ds(d * LANES, LANES)] = accs[d]

    start(0, rows0_v, sem0)
    start(1, rows1_v, sem1)

    for q in range(N_STAGE):
        pltpu.async_copy(feat_hbm.at[sidx_v.at[q]], srows_v, sem2)

        def pair_body(j, _, q=q):
            local = j - q * (CHUNKS_PER_STAGE // 2)
            c0 = 2 * j
            c1 = 2 * j + 1
            finish(c0, rows0_v, sem0)
            reduce_chunk(rows0_v, local * 2 * ITEMS_PER_CHUNK)

            @pl.when(c0 + 2 < CHUNKS_W)
            def _():
                start(c0 + 2, rows0_v, sem0)

            finish(c1, rows1_v, sem1)
            reduce_chunk(rows1_v, (local * 2 + 1) * ITEMS_PER_CHUNK)

            @pl.when(c1 + 2 < CHUNKS_W)
            def _():
                start(c1 + 2, rows1_v, sem1)

            return 0

        half = CHUNKS_PER_STAGE // 2
        lax.fori_loop(q * half, (q + 1) * half, pair_body, 0)
        pltpu.make_async_copy(feat_hbm.at[sidx_v.at[q]], srows_v, sem2).wait()
        pltpu.sync_copy(srows_v, self_out.at[pl.ds(out_base + q * CHUNK, CHUNK)])
        pltpu.sync_copy(stage_v, neigh_out.at[pl.ds(out_base + q * CHUNK, CHUNK)])


_sc_gather_mean = functools.partial(
    pl.kernel,
    out_type=(jax.ShapeDtypeStruct((B, D), jnp.float32),
              jax.ShapeDtypeStruct((B, D), jnp.float32)),
    mesh=plsc.VectorSubcoreMesh(core_axis_name="c", subcore_axis_name="s"),
    scratch_types=[
        pltpu.VMEM((CHUNKS_W, CHUNK), jnp.int32),
        pltpu.VMEM((N_STAGE, CHUNK), jnp.int32),
        pltpu.VMEM((CHUNK, D), jnp.float32),
        pltpu.VMEM((CHUNK, D), jnp.float32),
        pltpu.VMEM((CHUNK, D), jnp.float32),
        pltpu.VMEM((CHUNK, D), jnp.float32),
        pltpu.SemaphoreType.DMA,
        pltpu.SemaphoreType.DMA,
        pltpu.SemaphoreType.DMA,
    ],
)(_sc_body)


def _tc_head(self_feats, neigh_sum, Wt, Wb_perm, b1):
    BB = 2048

    def mm(self_ref, neigh_ref, wt_ref, wb_ref, b_ref, o_ref):
        s = self_ref[...]
        n = neigh_ref[...] * (1.0 / DEG)
        y = jnp.dot(s, wt_ref[...], preferred_element_type=jnp.float32,
                    precision=lax.Precision.HIGHEST)
        y = y + jnp.dot(n, wb_ref[...], preferred_element_type=jnp.float32,
                        precision=lax.Precision.HIGHEST)
        y = y + b_ref[...]
        o_ref[...] = jnp.maximum(y, 0.0)

    return pl.pallas_call(
        mm,
        grid=(B // BB,),
        in_specs=[
            pl.BlockSpec((BB, D), lambda i: (i, 0)),
            pl.BlockSpec((BB, D), lambda i: (i, 0)),
            pl.BlockSpec((D, D), lambda i: (0, 0)),
            pl.BlockSpec((D, D), lambda i: (0, 0)),
            pl.BlockSpec((1, D), lambda i: (0, 0)),
        ],
        out_specs=pl.BlockSpec((BB, D), lambda i: (i, 0)),
        out_shape=jax.ShapeDtypeStruct((B, D), jnp.float32),
    )(self_feats, neigh_sum, Wt, Wb_perm, b1.reshape(1, D))


def kernel(nodes, neigh_index, features, W1, b1):
    nodes2d = nodes.astype(jnp.int32).reshape(B // CHUNK, CHUNK)
    neigh2d = neigh_index.astype(jnp.int32).reshape(B * DEG // CHUNK, CHUNK)
    self_feats, neigh_sum = _sc_gather_mean(nodes2d, neigh2d, features)
    return _tc_head(self_feats, neigh_sum, W1[:D], W1[D:], b1)


# speedup vs baseline: 12.2818x; 1.2339x over previous
import functools

import jax
import jax.numpy as jnp
from jax import lax
from jax.experimental import pallas as pl
from jax.experimental.pallas import tpu as pltpu
from jax.experimental.pallas import tpu_sc as plsc

B = 16384
DEG = 32
D = 128
LANES = 16
NW = 32
NB_W = B // NW
CHUNK = 128
ITEMS_PER_CHUNK = CHUNK // DEG
CHUNKS_W = NB_W * DEG // CHUNK
N_STAGE = NB_W // CHUNK
CHUNKS_PER_STAGE = CHUNKS_W // N_STAGE


NBUF = 4


def _sc_body(nodes_hbm, neigh_hbm, feat_hbm, self_out, neigh_out,
             nidx_v, sidx_v, rows0_v, rows1_v, rows2_v, rows3_v,
             srows_v, stage_v, sem0, sem1, sem2, sem3, semself):
    rows_bufs = (rows0_v, rows1_v, rows2_v, rows3_v)
    sems = (sem0, sem1, sem2, sem3)
    wid = lax.axis_index("s") * 2 + lax.axis_index("c")
    pltpu.sync_copy(neigh_hbm.at[pl.ds(wid * CHUNKS_W, CHUNKS_W)], nidx_v)
    pltpu.sync_copy(nodes_hbm.at[pl.ds(wid * N_STAGE, N_STAGE)], sidx_v)
    out_base = wid * NB_W

    def start(c, rows, sem):
        pltpu.async_copy(feat_hbm.at[nidx_v.at[c]], rows, sem)

    def finish(c, rows, sem):
        pltpu.make_async_copy(feat_hbm.at[nidx_v.at[c]], rows, sem).wait()

    def row_vals(rows, r):
        return tuple(rows[r, pl.ds(d * LANES, LANES)]
                     for d in range(D // LANES))

    def reduce_chunk(rows, stage_base):
        for i in range(ITEMS_PER_CHUNK):
            base_r = i * DEG
            v0 = row_vals(rows, base_r)
            v1 = row_vals(rows, base_r + 1)
            accs = tuple(v0[d] + v1[d] for d in range(D // LANES))

            def add_body(k2, acc, base_r=base_r, rows=rows):
                r = base_r + 2 * k2
                va = row_vals(rows, r)
                vb = row_vals(rows, r + 1)
                return tuple(acc[d] + (va[d] + vb[d])
                             for d in range(D // LANES))

            accs = lax.fori_loop(1, DEG // 2, add_body, accs)
            for d in range(D // LANES):
                stage_v[stage_base + i, pl.ds(d * LANES, LANES)] = accs[d]

    for b in range(NBUF):
        start(b, rows_bufs[b], sems[b])

    for q in range(N_STAGE):
        pltpu.async_copy(feat_hbm.at[sidx_v.at[q]], srows_v, semself)

        def group_body(j, _, q=q):
            local = j - q * (CHUNKS_PER_STAGE // NBUF)
            for b in range(NBUF):
                c = NBUF * j + b
                finish(c, rows_bufs[b], sems[b])
                reduce_chunk(rows_bufs[b],
                             (local * NBUF + b) * ITEMS_PER_CHUNK)

                @pl.when(c + NBUF < CHUNKS_W)
                def _(c=c, b=b):
                    start(c + NBUF, rows_bufs[b], sems[b])

            return 0

        grp = CHUNKS_PER_STAGE // NBUF
        lax.fori_loop(q * grp, (q + 1) * grp, group_body, 0)
        pltpu.make_async_copy(feat_hbm.at[sidx_v.at[q]], srows_v, semself).wait()
        pltpu.sync_copy(srows_v, self_out.at[pl.ds(out_base + q * CHUNK, CHUNK)])
        pltpu.sync_copy(stage_v, neigh_out.at[pl.ds(out_base + q * CHUNK, CHUNK)])


_sc_gather_mean = functools.partial(
    pl.kernel,
    out_type=(jax.ShapeDtypeStruct((B, D), jnp.float32),
              jax.ShapeDtypeStruct((B, D), jnp.float32)),
    mesh=plsc.VectorSubcoreMesh(core_axis_name="c", subcore_axis_name="s"),
    scratch_types=[
        pltpu.VMEM((CHUNKS_W, CHUNK), jnp.int32),
        pltpu.VMEM((N_STAGE, CHUNK), jnp.int32),
        pltpu.VMEM((CHUNK, D), jnp.float32),
        pltpu.VMEM((CHUNK, D), jnp.float32),
        pltpu.VMEM((CHUNK, D), jnp.float32),
        pltpu.VMEM((CHUNK, D), jnp.float32),
        pltpu.VMEM((CHUNK, D), jnp.float32),
        pltpu.VMEM((CHUNK, D), jnp.float32),
        pltpu.SemaphoreType.DMA,
        pltpu.SemaphoreType.DMA,
        pltpu.SemaphoreType.DMA,
        pltpu.SemaphoreType.DMA,
        pltpu.SemaphoreType.DMA,
    ],
)(_sc_body)


def _tc_head(self_feats, neigh_sum, Wt, Wb_perm, b1):
    BB = 2048

    def mm(self_ref, neigh_ref, wt_ref, wb_ref, b_ref, o_ref):
        s = self_ref[...]
        n = neigh_ref[...] * (1.0 / DEG)
        y = jnp.dot(s, wt_ref[...], preferred_element_type=jnp.float32,
                    precision=lax.Precision.HIGHEST)
        y = y + jnp.dot(n, wb_ref[...], preferred_element_type=jnp.float32,
                        precision=lax.Precision.HIGHEST)
        y = y + b_ref[...]
        o_ref[...] = jnp.maximum(y, 0.0)

    return pl.pallas_call(
        mm,
        grid=(B // BB,),
        in_specs=[
            pl.BlockSpec((BB, D), lambda i: (i, 0)),
            pl.BlockSpec((BB, D), lambda i: (i, 0)),
            pl.BlockSpec((D, D), lambda i: (0, 0)),
            pl.BlockSpec((D, D), lambda i: (0, 0)),
            pl.BlockSpec((1, D), lambda i: (0, 0)),
        ],
        out_specs=pl.BlockSpec((BB, D), lambda i: (i, 0)),
        out_shape=jax.ShapeDtypeStruct((B, D), jnp.float32),
    )(self_feats, neigh_sum, Wt, Wb_perm, b1.reshape(1, D))


def kernel(nodes, neigh_index, features, W1, b1):
    nodes2d = nodes.astype(jnp.int32).reshape(B // CHUNK, CHUNK)
    neigh2d = neigh_index.astype(jnp.int32).reshape(B * DEG // CHUNK, CHUNK)
    self_feats, neigh_sum = _sc_gather_mean(nodes2d, neigh2d, features)
    return _tc_head(self_feats, neigh_sum, W1[:D], W1[D:], b1)


# speedup vs baseline: 12.4745x; 1.0157x over previous
import functools

import jax
import jax.numpy as jnp
from jax import lax
from jax.experimental import pallas as pl
from jax.experimental.pallas import tpu as pltpu
from jax.experimental.pallas import tpu_sc as plsc

B = 16384
DEG = 32
D = 128
LANES = 16
NW = 32
NB_W = B // NW
CHUNK = 64
ITEMS_PER_CHUNK = CHUNK // DEG
CHUNKS_W = NB_W * DEG // CHUNK
N_STAGE = NB_W // CHUNK
CHUNKS_PER_STAGE = CHUNKS_W // N_STAGE


NBUF = 8


def _sc_body(nodes_hbm, neigh_hbm, feat_hbm, self_out, neigh_out, *scr):
    nidx_v, sidx_v = scr[0], scr[1]
    rows_bufs = scr[2:2 + NBUF]
    srows_v = scr[2 + NBUF]
    stage_v = scr[3 + NBUF]
    sems = scr[4 + NBUF:4 + 2 * NBUF]
    semself = scr[4 + 2 * NBUF]
    wid = lax.axis_index("s") * 2 + lax.axis_index("c")
    pltpu.sync_copy(neigh_hbm.at[pl.ds(wid * CHUNKS_W, CHUNKS_W)], nidx_v)
    pltpu.sync_copy(nodes_hbm.at[pl.ds(wid * N_STAGE, N_STAGE)], sidx_v)
    out_base = wid * NB_W

    def start(c, rows, sem):
        pltpu.async_copy(feat_hbm.at[nidx_v.at[c]], rows, sem)

    def finish(c, rows, sem):
        pltpu.make_async_copy(feat_hbm.at[nidx_v.at[c]], rows, sem).wait()

    def row_vals(rows, r):
        return tuple(rows[r, pl.ds(d * LANES, LANES)]
                     for d in range(D // LANES))

    def reduce_chunk(rows, stage_base):
        for i in range(ITEMS_PER_CHUNK):
            base_r = i * DEG
            v0 = row_vals(rows, base_r)
            v1 = row_vals(rows, base_r + 1)
            accs = tuple(v0[d] + v1[d] for d in range(D // LANES))

            def add_body(k2, acc, base_r=base_r, rows=rows):
                r = base_r + 2 * k2
                va = row_vals(rows, r)
                vb = row_vals(rows, r + 1)
                return tuple(acc[d] + (va[d] + vb[d])
                             for d in range(D // LANES))

            accs = lax.fori_loop(1, DEG // 2, add_body, accs)
            for d in range(D // LANES):
                stage_v[stage_base + i, pl.ds(d * LANES, LANES)] = accs[d]

    for b in range(NBUF):
        start(b, rows_bufs[b], sems[b])

    grp = CHUNKS_PER_STAGE // NBUF

    def stage_body(q, _):
        pltpu.async_copy(feat_hbm.at[sidx_v.at[q]], srows_v, semself)

        def group_body(j, _):
            local = j - q * grp
            for b in range(NBUF):
                c = NBUF * j + b
                finish(c, rows_bufs[b], sems[b])
                reduce_chunk(rows_bufs[b],
                             (local * NBUF + b) * ITEMS_PER_CHUNK)

                @pl.when(c + NBUF < CHUNKS_W)
                def _(c=c, b=b):
                    start(c + NBUF, rows_bufs[b], sems[b])

            return 0

        lax.fori_loop(q * grp, (q + 1) * grp, group_body, 0)
        pltpu.make_async_copy(feat_hbm.at[sidx_v.at[q]], srows_v, semself).wait()
        pltpu.sync_copy(srows_v, self_out.at[pl.ds(out_base + q * CHUNK, CHUNK)])
        pltpu.sync_copy(stage_v, neigh_out.at[pl.ds(out_base + q * CHUNK, CHUNK)])
        return 0

    lax.fori_loop(0, N_STAGE, stage_body, 0)


_sc_gather_mean = functools.partial(
    pl.kernel,
    out_type=(jax.ShapeDtypeStruct((B, D), jnp.float32),
              jax.ShapeDtypeStruct((B, D), jnp.float32)),
    mesh=plsc.VectorSubcoreMesh(core_axis_name="c", subcore_axis_name="s"),
    scratch_types=(
        [pltpu.VMEM((CHUNKS_W, CHUNK), jnp.int32),
         pltpu.VMEM((N_STAGE, CHUNK), jnp.int32)]
        + [pltpu.VMEM((CHUNK, D), jnp.float32)
           for _ in range(NBUF)]
        + [pltpu.VMEM((CHUNK, D), jnp.float32),
           pltpu.VMEM((CHUNK, D), jnp.float32)]
        + [pltpu.SemaphoreType.DMA for _ in range(NBUF + 1)]
    ),
)(_sc_body)


def _tc_head(self_feats, neigh_sum, Wt, Wb_perm, b1):
    BB = 2048

    def mm(self_ref, neigh_ref, wt_ref, wb_ref, b_ref, o_ref):
        s = self_ref[...]
        n = neigh_ref[...] * (1.0 / DEG)
        y = jnp.dot(s, wt_ref[...], preferred_element_type=jnp.float32,
                    precision=lax.Precision.HIGHEST)
        y = y + jnp.dot(n, wb_ref[...], preferred_element_type=jnp.float32,
                        precision=lax.Precision.HIGHEST)
        y = y + b_ref[...]
        o_ref[...] = jnp.maximum(y, 0.0)

    return pl.pallas_call(
        mm,
        grid=(B // BB,),
        in_specs=[
            pl.BlockSpec((BB, D), lambda i: (i, 0)),
            pl.BlockSpec((BB, D), lambda i: (i, 0)),
            pl.BlockSpec((D, D), lambda i: (0, 0)),
            pl.BlockSpec((D, D), lambda i: (0, 0)),
            pl.BlockSpec((1, D), lambda i: (0, 0)),
        ],
        out_specs=pl.BlockSpec((BB, D), lambda i: (i, 0)),
        out_shape=jax.ShapeDtypeStruct((B, D), jnp.float32),
    )(self_feats, neigh_sum, Wt, Wb_perm, b1.reshape(1, D))


def kernel(nodes, neigh_index, features, W1, b1):
    nodes2d = nodes.astype(jnp.int32).reshape(B // CHUNK, CHUNK)
    neigh2d = neigh_index.astype(jnp.int32).reshape(B * DEG // CHUNK, CHUNK)
    self_feats, neigh_sum = _sc_gather_mean(nodes2d, neigh2d, features)
    return _tc_head(self_feats, neigh_sum, W1[:D], W1[D:], b1)


# speedup vs baseline: 13.0643x; 1.0473x over previous
import functools

import jax
import jax.numpy as jnp
from jax import lax
from jax.experimental import pallas as pl
from jax.experimental.pallas import tpu as pltpu
from jax.experimental.pallas import tpu_sc as plsc

B = 16384
DEG = 32
D = 128
LANES = 16
NW = 32
NB_W = B // NW
CHUNK = 64
ITEMS_PER_CHUNK = CHUNK // DEG
CHUNKS_W = NB_W * DEG // CHUNK
N_STAGE = NB_W // CHUNK
CHUNKS_PER_STAGE = CHUNKS_W // N_STAGE


NBUF = 8


def _sc_body(nodes_hbm, neigh_hbm, feat_hbm, self_out, neigh_out, *scr):
    nidx_v, sidx_v = scr[0], scr[1]
    rows_bufs = scr[2:2 + NBUF]
    srows_v = scr[2 + NBUF]
    stage_v = scr[3 + NBUF]
    sems = scr[4 + NBUF:4 + 2 * NBUF]
    semself = scr[4 + 2 * NBUF]
    wid = lax.axis_index("s") * 2 + lax.axis_index("c")
    pltpu.sync_copy(neigh_hbm.at[pl.ds(wid * CHUNKS_W, CHUNKS_W)], nidx_v)
    pltpu.sync_copy(nodes_hbm.at[pl.ds(wid * N_STAGE, N_STAGE)], sidx_v)
    out_base = wid * NB_W

    def start(c, rows, sem):
        pltpu.async_copy(feat_hbm.at[nidx_v.at[c]], rows, sem)

    def finish(c, rows, sem):
        pltpu.make_async_copy(feat_hbm.at[nidx_v.at[c]], rows, sem).wait()

    def row_vals(rows, r):
        return tuple(rows[r, pl.ds(d * LANES, LANES)]
                     for d in range(D // LANES))

    def reduce_chunk(rows, stage_base):
        for i in range(ITEMS_PER_CHUNK):
            base_r = i * DEG
            v0 = row_vals(rows, base_r)
            v1 = row_vals(rows, base_r + 1)
            accs = tuple(v0[d] + v1[d] for d in range(D // LANES))

            def add_body(k2, acc, base_r=base_r, rows=rows):
                r = base_r + 2 * k2
                va = row_vals(rows, r)
                vb = row_vals(rows, r + 1)
                return tuple(acc[d] + (va[d] + vb[d])
                             for d in range(D // LANES))

            accs = lax.fori_loop(1, DEG // 2, add_body, accs)
            for d in range(D // LANES):
                stage_v[stage_base + i, pl.ds(d * LANES, LANES)] = accs[d]

    for b in range(NBUF):
        start(b, rows_bufs[b], sems[b])

    grp = CHUNKS_PER_STAGE // NBUF

    def stage_body(q, _):
        pltpu.async_copy(feat_hbm.at[sidx_v.at[q]], srows_v, semself)

        def group_body(j, _):
            local = j - q * grp
            for b in range(NBUF):
                c = NBUF * j + b
                finish(c, rows_bufs[b], sems[b])
                reduce_chunk(rows_bufs[b],
                             (local * NBUF + b) * ITEMS_PER_CHUNK)

                @pl.when(c + NBUF < CHUNKS_W)
                def _(c=c, b=b):
                    start(c + NBUF, rows_bufs[b], sems[b])

            return 0

        lax.fori_loop(q * grp, (q + 1) * grp, group_body, 0)
        pltpu.make_async_copy(feat_hbm.at[sidx_v.at[q]], srows_v, semself).wait()
        pltpu.sync_copy(srows_v, self_out.at[pl.ds(out_base + q * CHUNK, CHUNK)])
        pltpu.sync_copy(stage_v, neigh_out.at[pl.ds(out_base + q * CHUNK, CHUNK)])
        return 0

    lax.fori_loop(0, N_STAGE, stage_body, 0)


_sc_gather_mean = functools.partial(
    pl.kernel,
    out_type=(jax.ShapeDtypeStruct((B, D), jnp.float32),
              jax.ShapeDtypeStruct((B, D), jnp.float32)),
    mesh=plsc.VectorSubcoreMesh(core_axis_name="c", subcore_axis_name="s"),
    scratch_types=(
        [pltpu.VMEM((CHUNKS_W, CHUNK), jnp.int32),
         pltpu.VMEM((N_STAGE, CHUNK), jnp.int32)]
        + [pltpu.VMEM((CHUNK, D), jnp.float32)
           for _ in range(NBUF)]
        + [pltpu.VMEM((CHUNK, D), jnp.float32),
           pltpu.VMEM((CHUNK, D), jnp.float32)]
        + [pltpu.SemaphoreType.DMA for _ in range(NBUF + 1)]
    ),
)(_sc_body)


def _tc_head(self_feats, neigh_sum, Wt, Wb_perm, b1):
    BB = 2048

    def mm(self_ref, neigh_ref, wt_ref, wb_ref, b_ref, o_ref):
        s = self_ref[...]
        n = neigh_ref[...] * (1.0 / DEG)
        y = jnp.dot(s, wt_ref[...], preferred_element_type=jnp.float32)
        y = y + jnp.dot(n, wb_ref[...], preferred_element_type=jnp.float32)
        y = y + b_ref[...]
        o_ref[...] = jnp.maximum(y, 0.0)

    return pl.pallas_call(
        mm,
        grid=(B // BB,),
        in_specs=[
            pl.BlockSpec((BB, D), lambda i: (i, 0)),
            pl.BlockSpec((BB, D), lambda i: (i, 0)),
            pl.BlockSpec((D, D), lambda i: (0, 0)),
            pl.BlockSpec((D, D), lambda i: (0, 0)),
            pl.BlockSpec((1, D), lambda i: (0, 0)),
        ],
        out_specs=pl.BlockSpec((BB, D), lambda i: (i, 0)),
        out_shape=jax.ShapeDtypeStruct((B, D), jnp.float32),
    )(self_feats, neigh_sum, Wt, Wb_perm, b1.reshape(1, D))


def kernel(nodes, neigh_index, features, W1, b1):
    nodes2d = nodes.astype(jnp.int32).reshape(B // CHUNK, CHUNK)
    neigh2d = neigh_index.astype(jnp.int32).reshape(B * DEG // CHUNK, CHUNK)
    self_feats, neigh_sum = _sc_gather_mean(nodes2d, neigh2d, features)
    return _tc_head(self_feats, neigh_sum, W1[:D], W1[D:], b1)
